# Initial kernel scaffold; baseline (speedup 1.0000x reference)
import jax
import jax.numpy as jnp
from jax.experimental import pallas as pl


def kernel(x, context_map, attention_weights_map, W_node, b_node, W_ctx, b_ctx, W_upd, b_upd, kan_base_w, kan_spline_w, kan_grid):
    raise NotImplementedError("write your pallas kernel here")


# speedup vs baseline: 1.4292x; 1.4292x over previous
import functools

import jax
import jax.numpy as jnp
from jax.experimental import pallas as pl

N = 10000
C = 16
F = 256
H = 256
O = 256
GRID = 3
ORDER = 2
NB = GRID + ORDER


def _dot_t(a, w):
    return jax.lax.dot_general(
        a, w, (((1,), (1,)), ((), ())), preferred_element_type=jnp.float32
    )


def _fused_kernel(cm_ref, aw_ref, x_ref, wn_ref, bn_ref, wc_ref, bc_ref,
                  wu_ref, bu_ref, wb_ref, wsp_ref, grid_ref, out_ref):
    aw = aw_ref[...] * (1.0 / C)
    cr = cm_ref[:, 0, :] * aw[:, 0:1]
    for c in range(1, C):
        cr = cr + cm_ref[:, c, :] * aw[:, c:c + 1]
    am = jnp.sum(aw, axis=1, keepdims=True)

    h = _dot_t(x_ref[...], wn_ref[...]) + bn_ref[...][None, :]
    h = h + _dot_t(cr, wc_ref[...]) + am * bc_ref[...][None, :]
    u = _dot_t(h, wu_ref[...]) + bu_ref[...][None, :]

    base = _dot_t(u * jax.nn.sigmoid(u), wb_ref[...])

    t = [grid_ref[0:1, j:j + 1] for j in range(GRID + 2 * ORDER + 1)]
    bases = [jnp.where((u >= t[j]) & (u < t[j + 1]), 1.0, 0.0)
             for j in range(GRID + 2 * ORDER)]
    for k in range(1, ORDER + 1):
        nxt = []
        for j in range(len(bases) - 1):
            left = (u - t[j]) / (t[j + k] - t[j]) * bases[j]
            right = (t[j + k + 1] - u) / (t[j + k + 1] - t[j + 1]) * bases[j + 1]
            nxt.append(left + right)
        bases = nxt
    acc = base
    for j in range(NB):
        acc = acc + _dot_t(bases[j], wsp_ref[j])
    out_ref[...] = acc


@functools.partial(jax.jit, static_argnames=())
def kernel(x, context_map, attention_weights_map, W_node, b_node, W_ctx,
           b_ctx, W_upd, b_upd, kan_base_w, kan_spline_w, kan_grid):
    Tn = 400
    grid = (N // Tn,)
    wsp = jnp.transpose(kan_spline_w, (2, 0, 1))

    full = lambda *s: pl.BlockSpec(s, lambda i: (0,) * len(s))
    return pl.pallas_call(
        _fused_kernel,
        grid=grid,
        in_specs=[
            pl.BlockSpec((Tn, C, F), lambda i: (i, 0, 0)),
            pl.BlockSpec((Tn, C), lambda i: (i, 0)),
            pl.BlockSpec((Tn, F), lambda i: (i, 0)),
            full(H, F), full(H), full(H, F), full(H),
            full(O, H), full(O), full(O, O), full(NB, O, O),
            full(O, GRID + 2 * ORDER + 1),
        ],
        out_specs=pl.BlockSpec((Tn, O), lambda i: (i, 0)),
        out_shape=jax.ShapeDtypeStruct((N, O), jnp.float32),
    )(context_map, attention_weights_map, x, W_node, b_node, W_ctx, b_ctx,
      W_upd, b_upd, kan_base_w, wsp, kan_grid)
